# baseline (speedup 1.0000x reference)
import functools

import jax
import jax.numpy as jnp
from jax import lax
from jax.experimental import pallas as pl
from jax.experimental.pallas import tpu as pltpu
from jax.experimental.pallas import tpu_sc as plsc

N_NODES = 10000
D = 128
DC = D // 2
E = 320000
L = 16
NW = 32
EPW = E // NW
CH = 80
NCHUNKS = EPW // CH
RPT = N_NODES // 16
RB = 125
RU = 5


def _edge_kernel(z, srci, dsti, out, sv, dv, arows, brows, outv, nbuf, pbuf,
                 shared, sems):
    cid = lax.axis_index("c")
    sid = lax.axis_index("s")
    wid = sid * 2 + cid
    base = pl.multiple_of(wid * EPW, 8)

    himask = jnp.int32(-65536)
    magic = jnp.int32(0x5F3759DF)

    def stage_batch(batch, carry):
        r0 = sid * RPT + batch * RB
        pltpu.sync_copy(z.at[pl.ds(r0, RB)], nbuf)

        def rows_body(i, carry2):
            for u in range(RU):
                r = i * RU + u
                xs = [nbuf[r, pl.ds(k * L, L)] for k in range(D // L)]
                sq = xs[0] * xs[0]
                for k in range(1, D // L):
                    sq = sq + xs[k] * xs[k]
                s = jnp.sum(sq)
                s16 = jnp.zeros((L,), jnp.float32) + jnp.maximum(s, 1e-12)
                y = plsc.bitcast(magic - (plsc.bitcast(s16, jnp.int32) >> 1),
                                 jnp.float32)
                for _ in range(3):
                    y = y * (1.5 - 0.5 * s16 * y * y)
                for k in range(DC // L):
                    pk = plsc.pack(xs[k] * y, xs[k + DC // L] * y,
                                   format=plsc.PackFormat.INTERLEAVED)
                    pbuf[r, pl.ds(k * L, L)] = plsc.bitcast(pk, jnp.int32)
            return carry2

        lax.fori_loop(0, RB // RU, rows_body, 0)
        pltpu.sync_copy(pbuf, shared.at[pl.ds(r0, RB)])
        return carry

    lax.fori_loop(0, RPT // RB, stage_batch, 0)

    pltpu.sync_copy(srci.at[pl.ds(base, EPW)], sv)
    pltpu.sync_copy(dsti.at[pl.ds(base, EPW)], dv)

    plsc.subcore_barrier()

    def issue(g, b):
        coff = pl.multiple_of(g * CH, 8)
        roff = pl.multiple_of(b * CH, 8)
        pltpu.async_copy(shared.at[sv.at[pl.ds(coff, CH)]],
                         arows.at[pl.ds(roff, CH)], sems.at[b])
        pltpu.async_copy(shared.at[dv.at[pl.ds(coff, CH)]],
                         brows.at[pl.ds(roff, CH)], sems.at[b])

    def wait(g, b):
        coff = pl.multiple_of(g * CH, 8)
        roff = pl.multiple_of(b * CH, 8)
        pltpu.make_async_copy(shared.at[sv.at[pl.ds(coff, CH)]],
                              arows.at[pl.ds(roff, CH)], sems.at[b]).wait()
        pltpu.make_async_copy(shared.at[dv.at[pl.ds(coff, CH)]],
                              brows.at[pl.ds(roff, CH)], sems.at[b]).wait()

    issue(0, 0)

    def chunk_body(g, carry):
        b = g & 1

        @pl.when(g + 1 < NCHUNKS)
        def _():
            issue(g + 1, 1 - b)

        wait(g, b)
        rbase = b * CH

        for e0 in range(0, CH, L):
            ev = rbase + e0 + lax.iota(jnp.int32, L)
            dv_ = lax.iota(jnp.int32, L)
            acc32 = jnp.zeros((2 * L,), jnp.bfloat16)
            for _t in range(DC):
                ap = plsc.load_gather(arows, [ev, dv_])
                bp = plsc.load_gather(brows, [ev, dv_])
                av = plsc.bitcast(ap, jnp.bfloat16)
                bv = plsc.bitcast(bp, jnp.bfloat16)
                acc32 = acc32 + av * bv
                dv_ = (dv_ + 1) & (DC - 1)
            acci = plsc.bitcast(acc32, jnp.int32)
            acc = (plsc.bitcast(acci << 16, jnp.float32)
                   + plsc.bitcast(acci & himask, jnp.float32))
            outv[pl.ds(g * CH + e0, L)] = 1.0 / (1.0 + jnp.exp(-acc))
        return carry

    lax.fori_loop(0, NCHUNKS, chunk_body, 0)

    pltpu.sync_copy(outv, out.at[pl.ds(base, EPW)])


def _make_sc_call():
    mesh = plsc.VectorSubcoreMesh(core_axis_name="c", subcore_axis_name="s")
    return functools.partial(
        pl.kernel,
        mesh=mesh,
        compiler_params=pltpu.CompilerParams(
            needs_layout_passes=False, use_tc_tiling_on_sc=False),
        out_type=jax.ShapeDtypeStruct((E,), jnp.float32),
        scratch_types=[
            pltpu.VMEM((EPW,), jnp.int32),
            pltpu.VMEM((EPW,), jnp.int32),
            pltpu.VMEM((2 * CH, DC), jnp.int32),
            pltpu.VMEM((2 * CH, DC), jnp.int32),
            pltpu.VMEM((EPW,), jnp.float32),
            pltpu.VMEM((RB, D), jnp.float32),
            pltpu.VMEM((RB, DC), jnp.int32),
            pltpu.VMEM_SHARED((N_NODES, DC), jnp.int32),
            pltpu.SemaphoreType.DMA((2,)),
        ],
    )(_edge_kernel)


def kernel(z, edge_index):
    src = edge_index[0]
    dst = edge_index[1]
    return _make_sc_call()(z, src, dst)


# speedup vs baseline: 1.2239x; 1.2239x over previous
import functools

import jax
import jax.numpy as jnp
from jax import lax
from jax.experimental import pallas as pl
from jax.experimental.pallas import tpu as pltpu
from jax.experimental.pallas import tpu_sc as plsc

N_NODES = 10000
D = 128
DC = D // 2
E = 320000
L = 16
NW = 32
EPW = E // NW
CH = 80
NCHUNKS = EPW // CH


def _normalize_body(z_ref, o_ref):
    x = z_ref[...]
    ss = jnp.sum(x * x, axis=1, keepdims=True)
    xn = x * lax.rsqrt(jnp.maximum(ss, 1e-12))
    xb = xn.astype(jnp.bfloat16)
    lo = lax.bitcast_convert_type(xb[:, :DC], jnp.uint16).astype(jnp.uint32)
    hi = lax.bitcast_convert_type(xb[:, DC:], jnp.uint16).astype(jnp.uint32)
    o_ref[...] = lax.bitcast_convert_type(lo | (hi << 16), jnp.int32)


def _normalize_pack(z):
    n = z.shape[0]
    blk = 2000
    return pl.pallas_call(
        _normalize_body,
        grid=(n // blk,),
        in_specs=[pl.BlockSpec((blk, D), lambda i: (i, 0))],
        out_specs=pl.BlockSpec((blk, DC), lambda i: (i, 0)),
        out_shape=jax.ShapeDtypeStruct((n, DC), jnp.int32),
    )(z)


def _edge_kernel(zn, srci, dsti, out, sv, dv, arows, brows, outv, shared, sems):
    cid = lax.axis_index("c")
    sid = lax.axis_index("s")
    wid = sid * 2 + cid
    base = pl.multiple_of(wid * EPW, 8)

    rpt = N_NODES // 16
    pltpu.sync_copy(zn.at[pl.ds(sid * rpt, rpt)],
                    shared.at[pl.ds(sid * rpt, rpt)])

    pltpu.sync_copy(srci.at[pl.ds(base, EPW)], sv)
    pltpu.sync_copy(dsti.at[pl.ds(base, EPW)], dv)

    plsc.subcore_barrier()

    himask = jnp.int32(-65536)

    def issue(g, b):
        coff = pl.multiple_of(g * CH, 8)
        roff = pl.multiple_of(b * CH, 8)
        pltpu.async_copy(shared.at[sv.at[pl.ds(coff, CH)]],
                         arows.at[pl.ds(roff, CH)], sems.at[b])
        pltpu.async_copy(shared.at[dv.at[pl.ds(coff, CH)]],
                         brows.at[pl.ds(roff, CH)], sems.at[b])

    def wait(g, b):
        coff = pl.multiple_of(g * CH, 8)
        roff = pl.multiple_of(b * CH, 8)
        pltpu.make_async_copy(shared.at[sv.at[pl.ds(coff, CH)]],
                              arows.at[pl.ds(roff, CH)], sems.at[b]).wait()
        pltpu.make_async_copy(shared.at[dv.at[pl.ds(coff, CH)]],
                              brows.at[pl.ds(roff, CH)], sems.at[b]).wait()

    issue(0, 0)
    issue(1, 1)
    issue(2, 2)

    def chunk_body(g, carry):
        b = g & 3

        @pl.when(g + 3 < NCHUNKS)
        def _():
            issue(g + 3, (g + 3) & 3)

        wait(g, b)
        rbase = b * CH

        for e0 in range(0, CH, L):
            ev = rbase + e0 + lax.iota(jnp.int32, L)
            dv_ = lax.iota(jnp.int32, L)
            acc32 = jnp.zeros((2 * L,), jnp.bfloat16)
            for _t in range(DC):
                ap = plsc.load_gather(arows, [ev, dv_])
                bp = plsc.load_gather(brows, [ev, dv_])
                av = plsc.bitcast(ap, jnp.bfloat16)
                bv = plsc.bitcast(bp, jnp.bfloat16)
                acc32 = acc32 + av * bv
                dv_ = (dv_ + 1) & (DC - 1)
            acci = plsc.bitcast(acc32, jnp.int32)
            acc = (plsc.bitcast(acci << 16, jnp.float32)
                   + plsc.bitcast(acci & himask, jnp.float32))
            outv[pl.ds(g * CH + e0, L)] = 1.0 / (1.0 + jnp.exp(-acc))
        return carry

    lax.fori_loop(0, NCHUNKS, chunk_body, 0)

    pltpu.sync_copy(outv, out.at[pl.ds(base, EPW)])


def _make_sc_call():
    mesh = plsc.VectorSubcoreMesh(core_axis_name="c", subcore_axis_name="s")
    return functools.partial(
        pl.kernel,
        mesh=mesh,
        compiler_params=pltpu.CompilerParams(
            needs_layout_passes=False, use_tc_tiling_on_sc=False),
        out_type=jax.ShapeDtypeStruct((E,), jnp.float32),
        scratch_types=[
            pltpu.VMEM((EPW,), jnp.int32),
            pltpu.VMEM((EPW,), jnp.int32),
            pltpu.VMEM((4 * CH, DC), jnp.int32),
            pltpu.VMEM((4 * CH, DC), jnp.int32),
            pltpu.VMEM((EPW,), jnp.float32),
            pltpu.VMEM_SHARED((N_NODES, DC), jnp.int32),
            pltpu.SemaphoreType.DMA((4,)),
        ],
    )(_edge_kernel)


def kernel(z, edge_index):
    zn = _normalize_pack(z)
    src = edge_index[0]
    dst = edge_index[1]
    return _make_sc_call()(zn, src, dst)
